# baseline (speedup 1.0000x reference)
import functools

import jax
import jax.numpy as jnp
from jax import lax
from jax.experimental import pallas as pl
from jax.experimental.pallas import tpu as pltpu
from jax.experimental.pallas import tpu_sc as plsc

VOCAB = 1000000
D = 64
NB = 16384
H = 50
NC, NS = 2, 16
NW = NC * NS
BPW = NB // NW
NBLK = BPW // 128
NCHUNK = NBLK * H
NSLOT = 4


def _gather_body(idx_hbm, table_hbm, out_hbm, xv, idxv, rows, tr,
                 sg0, sg1, sg2, sg3, sw0, sw1, sw2, sw3):
    wid = lax.axis_index("s") * NC + lax.axis_index("c")

    pltpu.sync_copy(idx_hbm.at[wid], xv)

    sem_g = (sg0, sg1, sg2, sg3)
    sem_w = (sw0, sw1, sw2, sw3)
    iota = lax.iota(jnp.int32, 16)

    def stage_idx(c, slot):
        bblk = c // H
        h = c % H
        base = bblk * (128 * H) + h
        for g in range(8):
            src = (iota + (g * 16)) * H + base
            idxv.at[slot][pl.ds(g * 16, 16)] = plsc.load_gather(xv, [src])

    def fire_gather(c, slot):
        pltpu.async_copy(table_hbm.at[idxv.at[slot]], rows.at[slot],
                         sem_g[slot])

    def wait_gather(slot):
        pltpu.make_async_copy(table_hbm.at[idxv.at[slot]], rows.at[slot],
                              sem_g[slot]).wait()

    dhi_c = [(iota + g * 16) // 8 for g in range(4)]
    dlo_c = [(iota + g * 16) % 8 for g in range(4)]

    def transpose(slot):
        src = rows.at[slot]
        dst = tr.at[slot]

        def ld(j):
            row = src.at[j]
            return [row[pl.ds(g * 16, 16)] for g in range(4)]

        def st(js, vals):
            for g in range(4):
                plsc.store_scatter(dst, [dhi_c[g], dlo_c[g], js], vals[g])

        @pl.loop(0, 128, step=8, init_carry=iota * 0)
        def _(j0, jvec):
            v0 = ld(j0)
            v1 = ld(j0 + 1)
            for jo in range(2, 8):
                nxt = ld(j0 + jo)
                st(jvec + (jo - 2), v0)
                v0, v1 = v1, nxt
            st(jvec + 6, v0)
            st(jvec + 7, v1)
            return jvec + 8

    def fire_write(c, slot):
        bblk = c // H
        h = c % H
        gbblk = wid * NBLK + bblk
        pltpu.async_copy(tr.at[slot, :, :, pl.ds(0, 128)],
                         out_hbm.at[h, :, gbblk], sem_w[slot])

    def drain_write(slot):
        pltpu.make_async_copy(tr.at[slot, :, :, pl.ds(0, 128)],
                              out_hbm.at[0, :, 0], sem_w[slot]).wait()

    for c in (0, 1, 2):
        stage_idx(c, c)
        fire_gather(c, c)

    @pl.loop(0, NCHUNK, step=NSLOT)
    def _(j):
        for s in range(NSLOT):
            c = j + s

            @pl.when(c + 3 < NCHUNK)
            def _():
                s3 = (s + 3) % NSLOT
                stage_idx(c + 3, s3)
                fire_gather(c + 3, s3)

            @pl.when(c >= NSLOT)
            def _():
                drain_write(s)

            wait_gather(s)
            transpose(s)
            fire_write(c, s)

    for s in range(NSLOT):
        drain_write(s)


@jax.jit
def _lookup(idx, table):
    mesh = plsc.VectorSubcoreMesh(core_axis_name="c", subcore_axis_name="s")
    f = functools.partial(
        pl.kernel,
        mesh=mesh,
        out_type=jax.ShapeDtypeStruct((H, 8, NB // 128, 8, 128),
                                      jnp.float32),
        scratch_types=[
            pltpu.VMEM((BPW * H,), jnp.int32),
            pltpu.VMEM((NSLOT, 128), jnp.int32),
            pltpu.VMEM((NSLOT, 128, D), jnp.float32),
            pltpu.VMEM((NSLOT, 8, 8, 133), jnp.float32),
            pltpu.SemaphoreType.DMA,
            pltpu.SemaphoreType.DMA,
            pltpu.SemaphoreType.DMA,
            pltpu.SemaphoreType.DMA,
            pltpu.SemaphoreType.DMA,
            pltpu.SemaphoreType.DMA,
            pltpu.SemaphoreType.DMA,
            pltpu.SemaphoreType.DMA,
        ],
        compiler_params=pltpu.CompilerParams(use_tc_tiling_on_sc=False,
                                             needs_layout_passes=False),
    )(_gather_body)
    return f(idx, table)


def kernel(x, embedding_table):
    idx = x.astype(jnp.int32).reshape(NW, BPW * H)
    p = _lookup(idx, embedding_table)
    return p.transpose(2, 4, 0, 1, 3).reshape(NB, H, D)


# speedup vs baseline: 1.0129x; 1.0129x over previous
import functools

import jax
import jax.numpy as jnp
from jax import lax
from jax.experimental import pallas as pl
from jax.experimental.pallas import tpu as pltpu
from jax.experimental.pallas import tpu_sc as plsc

VOCAB = 1000000
D = 64
NB = 16384
H = 50
NC, NS = 2, 16
NW = NC * NS
BPW = NB // NW
NBLK = BPW // 128
NCHUNK = NBLK * H
NSLOT = 4


def _gather_body(idx_hbm, table_hbm, out_hbm, xv, idxv, rows, tr,
                 sg0, sg1, sg2, sg3, sw0, sw1, sw2, sw3):
    wid = lax.axis_index("s") * NC + lax.axis_index("c")

    pltpu.sync_copy(idx_hbm.at[wid], xv)

    sem_g = (sg0, sg1, sg2, sg3)
    sem_w = (sw0, sw1, sw2, sw3)
    iota = lax.iota(jnp.int32, 16)

    def stage_idx(c, slot):
        bblk = c // H
        h = c % H
        base = bblk * (128 * H) + h
        for g in range(8):
            src = (iota + (g * 16)) * H + base
            idxv.at[slot][pl.ds(g * 16, 16)] = plsc.load_gather(xv, [src])

    def fire_gather(c, slot):
        pltpu.async_copy(table_hbm.at[idxv.at[slot]], rows.at[slot],
                         sem_g[slot])

    def wait_gather(slot):
        pltpu.make_async_copy(table_hbm.at[idxv.at[slot]], rows.at[slot],
                              sem_g[slot]).wait()

    dhi_c = [(iota + g * 16) // 8 for g in range(4)]
    dlo_c = [(iota + g * 16) % 8 for g in range(4)]

    def transpose(slot):
        src = rows.at[slot]
        dst = tr.at[slot]

        def ld(j):
            row = src.at[j]
            return [row[pl.ds(g * 16, 16)] for g in range(4)]

        def st(js, vals):
            for g in range(4):
                plsc.store_scatter(dst, [dhi_c[g], dlo_c[g], js], vals[g])

        @pl.loop(0, 128, step=8, init_carry=iota * 0)
        def _(j0, jvec):
            vals = ld(j0)
            for jo in range(1, 8):
                nxt = ld(j0 + jo)
                st(jvec + (jo - 1), vals)
                vals = nxt
            st(jvec + 7, vals)
            return jvec + 8

    def fire_write(c, slot):
        bblk = c // H
        h = c % H
        gbblk = wid * NBLK + bblk
        pltpu.async_copy(tr.at[slot, :, :, pl.ds(0, 128)],
                         out_hbm.at[h, :, gbblk], sem_w[slot])

    def drain_write(slot):
        pltpu.make_async_copy(tr.at[slot, :, :, pl.ds(0, 128)],
                              out_hbm.at[0, :, 0], sem_w[slot]).wait()

    for c in (0, 1, 2):
        stage_idx(c, c)
        fire_gather(c, c)

    @pl.loop(0, NCHUNK, step=NSLOT)
    def _(j):
        for s in range(NSLOT):
            c = j + s

            @pl.when(c + 3 < NCHUNK)
            def _():
                s3 = (s + 3) % NSLOT
                stage_idx(c + 3, s3)
                fire_gather(c + 3, s3)

            @pl.when(c >= NSLOT)
            def _():
                drain_write(s)

            wait_gather(s)
            transpose(s)
            fire_write(c, s)

    for s in range(NSLOT):
        drain_write(s)


@jax.jit
def _lookup(idx, table):
    mesh = plsc.VectorSubcoreMesh(core_axis_name="c", subcore_axis_name="s")
    f = functools.partial(
        pl.kernel,
        mesh=mesh,
        out_type=jax.ShapeDtypeStruct((H, 8, NB // 128, 8, 128),
                                      jnp.float32),
        scratch_types=[
            pltpu.VMEM((BPW * H,), jnp.int32),
            pltpu.VMEM((NSLOT, 128), jnp.int32),
            pltpu.VMEM((NSLOT, 128, D), jnp.float32),
            pltpu.VMEM((NSLOT, 8, 8, 133), jnp.float32),
            pltpu.SemaphoreType.DMA,
            pltpu.SemaphoreType.DMA,
            pltpu.SemaphoreType.DMA,
            pltpu.SemaphoreType.DMA,
            pltpu.SemaphoreType.DMA,
            pltpu.SemaphoreType.DMA,
            pltpu.SemaphoreType.DMA,
            pltpu.SemaphoreType.DMA,
        ],
        compiler_params=pltpu.CompilerParams(use_tc_tiling_on_sc=False,
                                             needs_layout_passes=False),
    )(_gather_body)
    return f(idx, table)


def kernel(x, embedding_table):
    idx = x.astype(jnp.int32).reshape(NW, BPW * H)
    p = _lookup(idx, embedding_table)
    return p.transpose(2, 4, 0, 1, 3).reshape(NB, H, D)
